# Initial kernel scaffold; baseline (speedup 1.0000x reference)
import functools

import jax
import jax.numpy as jnp
from jax import lax
from jax.experimental import pallas as pl
from jax.experimental.pallas import tpu as pltpu
from jax.experimental.pallas import tpu_sc as plsc

_N = 100000
_D = 128
_G = 1024


_TILE = 5000


def _rowdot_body(x_ref, w_ref, o_ref):
    w = w_ref[0, :]
    o_ref[...] = jnp.sum(x_ref[...] * w[None, :], axis=1)


def _rowdot(x, w_row):
    n = x.shape[0]
    return pl.pallas_call(
        _rowdot_body,
        grid=(n // _TILE,),
        in_specs=[
            pl.BlockSpec((_TILE, _D), lambda i: (i, 0)),
            pl.BlockSpec((1, _D), lambda i: (0, 0)),
        ],
        out_specs=pl.BlockSpec((_TILE,), lambda i: (i,)),
        out_shape=jax.ShapeDtypeStruct((n,), jnp.float32),
    )(x, w_row)


_NT = 16
_NPAD = 102400
_NPT = _NPAD // _NT
_NV = _NPT // 16
_GP = _G + 16
_BPT = _G // _NT


def _seg_body(s_hbm, ids_hbm, b_hbm, out_hbm,
              s_v, ids_v, acc_s, acc_c, shared, buf, pred_v, b_v):
    sid = lax.axis_index("s")
    base = sid * _NPT
    pltpu.sync_copy(s_hbm.at[pl.ds(base, _NPT)], s_v)
    pltpu.sync_copy(ids_hbm.at[pl.ds(base, _NPT + 16)], ids_v)

    zeros16 = jnp.zeros((16,), jnp.float32)

    def _zero(i, carry):
        acc_s[pl.ds(i * 16, 16)] = zeros16
        acc_c[pl.ds(i * 16, 16)] = zeros16
        return carry

    lax.fori_loop(0, _GP // 16, _zero, 0)

    lane = lax.iota(jnp.int32, 16)
    pos = lax.convert_element_type(lane, jnp.float32) + 1.0
    is15 = lane == 15

    def _step(j, carry):
        off = j * 16
        v = s_v[pl.ds(off, 16)]
        ids = ids_v[pl.ds(off, 16)]
        idn = ids_v[pl.ds(off + 1, 16)]
        c = plsc.cumsum(v)
        bnd = ids != idn
        m_add = bnd | is15
        m_sub = bnd & jnp.logical_not(is15)
        plsc.addupdate_scatter(acc_s, [ids], c, mask=m_add)
        plsc.addupdate_scatter(acc_s, [idn], -c, mask=m_sub)
        plsc.addupdate_scatter(acc_c, [ids], pos, mask=m_add)
        plsc.addupdate_scatter(acc_c, [idn], -pos, mask=m_sub)
        return carry

    lax.fori_loop(0, _NV, _step, 0)

    pltpu.sync_copy(acc_s, shared.at[sid, 0])
    pltpu.sync_copy(acc_c, shared.at[sid, 1])
    plsc.subcore_barrier()

    gb = sid * _BPT
    for t in range(_NT):
        pltpu.sync_copy(shared.at[t, 0, pl.ds(gb, _BPT)], buf.at[t, 0])
        pltpu.sync_copy(shared.at[t, 1, pl.ds(gb, _BPT)], buf.at[t, 1])
    pltpu.sync_copy(b_hbm, b_v)
    bvec = b_v[...]
    for k in range(_BPT // 16):
        ss = zeros16
        cc = zeros16
        for t in range(_NT):
            ss = ss + buf[t, 0, pl.ds(k * 16, 16)]
            cc = cc + buf[t, 1, pl.ds(k * 16, 16)]
        pred_v[pl.ds(k * 16, 16)] = ss / jnp.maximum(cc, 1.0) + bvec
    pltpu.sync_copy(pred_v, out_hbm.at[pl.ds(gb, _BPT)])


def _segment_mean_linear(s_pad, ids_pad, b16):
    mesh = plsc.VectorSubcoreMesh(
        core_axis_name="c", subcore_axis_name="s", num_cores=1)
    f = functools.partial(
        pl.kernel,
        mesh=mesh,
        out_type=jax.ShapeDtypeStruct((_G,), jnp.float32),
        scratch_types=[
            pltpu.VMEM((_NPT,), jnp.float32),
            pltpu.VMEM((_NPT + 16,), jnp.int32),
            pltpu.VMEM((_GP,), jnp.float32),
            pltpu.VMEM((_GP,), jnp.float32),
            pltpu.VMEM_SHARED((_NT, 2, _GP), jnp.float32),
            pltpu.VMEM((_NT, 2, _BPT), jnp.float32),
            pltpu.VMEM((_BPT,), jnp.float32),
            pltpu.VMEM((16,), jnp.float32),
        ],
    )(_seg_body)
    return f(s_pad, ids_pad, b16)


def kernel(x, batch, y, W, b):
    s = _rowdot(x, W.reshape(1, _D))
    s_pad = jnp.pad(s, (0, _NPAD - _N))
    ids = batch.astype(jnp.int32)
    ids_pad = jnp.pad(ids, (0, _NPAD + 16 - _N), constant_values=_G)
    b16 = jnp.broadcast_to(b.reshape(1), (16,)).astype(jnp.float32)
    pred = _segment_mean_linear(s_pad, ids_pad, b16).reshape(_G, 1)
    return (pred, y)


# speedup vs baseline: 5.1082x; 5.1082x over previous
import functools

import jax
import jax.numpy as jnp
from jax import lax
from jax.experimental import pallas as pl
from jax.experimental.pallas import tpu as pltpu
from jax.experimental.pallas import tpu_sc as plsc

_N = 100000
_D = 128
_G = 1024


_TILE = 5000


def _rowdot_body(x_ref, w_ref, o_ref):
    w = w_ref[0, :]
    o_ref[...] = jnp.sum(x_ref[...] * w[None, :], axis=1, keepdims=True)


def _rowdot(x, w_row):
    n = x.shape[0]
    return pl.pallas_call(
        _rowdot_body,
        grid=(n // _TILE,),
        in_specs=[
            pl.BlockSpec((_TILE, _D), lambda i: (i, 0)),
            pl.BlockSpec((1, _D), lambda i: (0, 0)),
        ],
        out_specs=pl.BlockSpec((_TILE, 1), lambda i: (i, 0)),
        out_shape=jax.ShapeDtypeStruct((n, 1), jnp.float32),
    )(x, w_row)


_NT = 16
_NPAD = 102400
_NPT = _NPAD // _NT
_NV = _NPT // 16
_GP = _G + 16
_BPT = _G // _NT


def _seg_body(s_hbm, ids_hbm, b_hbm, out_hbm, part_s_hbm, part_c_hbm,
              s_v, ids_v, acc_s, acc_c, buf_s, buf_c, pred_v, b_v):
    sid = lax.axis_index("s")
    base = sid * _NPT
    pltpu.sync_copy(s_hbm.at[pl.ds(base, _NPT)], s_v)
    pltpu.sync_copy(ids_hbm.at[pl.ds(base, _NPT + 16)], ids_v)

    zeros16 = jnp.zeros((16,), jnp.float32)

    def _zero(i, carry):
        acc_s[pl.ds(i * 16, 16)] = zeros16
        acc_c[pl.ds(i * 16, 16)] = zeros16
        return carry

    lax.fori_loop(0, _GP // 16, _zero, 0)

    lane = lax.iota(jnp.int32, 16)
    pos = lax.convert_element_type(lane, jnp.float32) + 1.0
    is15 = lane == 15

    def _step(j, carry):
        off = j * 16
        v = s_v[pl.ds(off, 16)]
        ids = ids_v[pl.ds(off, 16)]
        idn = ids_v[pl.ds(off + 1, 16)]
        c = jnp.cumsum(v)
        bnd = ids != idn
        m_add = bnd | is15
        m_sub = bnd & jnp.logical_not(is15)
        plsc.addupdate_scatter(acc_s, [ids], c, mask=m_add)
        plsc.addupdate_scatter(acc_s, [idn], -c, mask=m_sub)
        plsc.addupdate_scatter(acc_c, [ids], pos, mask=m_add)
        plsc.addupdate_scatter(acc_c, [idn], -pos, mask=m_sub)
        return carry

    lax.fori_loop(0, _NV, _step, 0)

    pltpu.sync_copy(acc_s, part_s_hbm.at[pl.ds(sid * _GP, _GP)])
    pltpu.sync_copy(acc_c, part_c_hbm.at[pl.ds(sid * _GP, _GP)])
    plsc.subcore_barrier()

    gb = sid * _BPT
    for t in range(_NT):
        pltpu.sync_copy(part_s_hbm.at[pl.ds(t * _GP + gb, _BPT)],
                        buf_s.at[pl.ds(t * _BPT, _BPT)])
        pltpu.sync_copy(part_c_hbm.at[pl.ds(t * _GP + gb, _BPT)],
                        buf_c.at[pl.ds(t * _BPT, _BPT)])
    pltpu.sync_copy(b_hbm, b_v)
    bvec = b_v[...]
    for k in range(_BPT // 16):
        ss = zeros16
        cc = zeros16
        for t in range(_NT):
            ss = ss + buf_s[pl.ds(t * _BPT + k * 16, 16)]
            cc = cc + buf_c[pl.ds(t * _BPT + k * 16, 16)]
        pred_v[pl.ds(k * 16, 16)] = ss / jnp.maximum(cc, 1.0) + bvec
    pltpu.sync_copy(pred_v, out_hbm.at[pl.ds(gb, _BPT)])


def _segment_mean_linear(s_pad, ids_pad, b16):
    mesh = plsc.VectorSubcoreMesh(
        core_axis_name="c", subcore_axis_name="s", num_cores=1)
    f = functools.partial(
        pl.kernel,
        mesh=mesh,
        compiler_params=pltpu.CompilerParams(needs_layout_passes=False),
        out_type=(
            jax.ShapeDtypeStruct((_G,), jnp.float32),
            jax.ShapeDtypeStruct((_NT * _GP,), jnp.float32),
            jax.ShapeDtypeStruct((_NT * _GP,), jnp.float32),
        ),
        scratch_types=[
            pltpu.VMEM((_NPT,), jnp.float32),
            pltpu.VMEM((_NPT + 16,), jnp.int32),
            pltpu.VMEM((_GP,), jnp.float32),
            pltpu.VMEM((_GP,), jnp.float32),
            pltpu.VMEM((_G,), jnp.float32),
            pltpu.VMEM((_G,), jnp.float32),
            pltpu.VMEM((_BPT,), jnp.float32),
            pltpu.VMEM((16,), jnp.float32),
        ],
    )(_seg_body)
    return f(s_pad, ids_pad, b16)[0]


def kernel(x, batch, y, W, b):
    s = _rowdot(x, W.reshape(1, _D)).reshape(_N)
    s_pad = jnp.pad(s, (0, _NPAD - _N))
    ids = batch.astype(jnp.int32)
    ids_pad = jnp.pad(ids, (0, _NPAD + 16 - _N), constant_values=_G)
    b16 = jnp.broadcast_to(b.reshape(1), (16,)).astype(jnp.float32)
    pred = _segment_mean_linear(s_pad, ids_pad, b16).reshape(_G, 1)
    return (pred, y)


# speedup vs baseline: 5.8891x; 1.1529x over previous
import functools

import jax
import jax.numpy as jnp
from jax import lax
from jax.experimental import pallas as pl
from jax.experimental.pallas import tpu as pltpu
from jax.experimental.pallas import tpu_sc as plsc

_N = 100000
_D = 128
_G = 1024


_TILE = 5000


def _rowdot_body(x_ref, w_ref, o_ref):
    w = w_ref[0, :]
    o_ref[...] = jnp.sum(x_ref[...] * w[None, :], axis=1, keepdims=True)


def _rowdot(x, w_row):
    n = x.shape[0]
    return pl.pallas_call(
        _rowdot_body,
        grid=(n // _TILE,),
        in_specs=[
            pl.BlockSpec((_TILE, _D), lambda i: (i, 0)),
            pl.BlockSpec((1, _D), lambda i: (0, 0)),
        ],
        out_specs=pl.BlockSpec((_TILE, 1), lambda i: (i, 0)),
        out_shape=jax.ShapeDtypeStruct((_NPAD, 1), jnp.float32),
    )(x, w_row)


_NT = 16
_NPAD = 102400
_NPT = _NPAD // _NT
_NV = _NPT // 16
_GP = _G + 16
_BPT = _G // _NT


_NTAIL = _NPAD + 16 - _N


def _seg_body(s_hbm, ids_hbm, tail_hbm, b_hbm, out_hbm, part_s_hbm, part_c_hbm,
              s_v, ids_v, acc_s, acc_c, buf_s, buf_c, pred_v, b_v, sem):
    sid = lax.axis_index("s")
    base = sid * _NPT
    pltpu.sync_copy(s_hbm.at[pl.ds(base, _NPT)], s_v)

    @pl.when(sid < _NT - 1)
    def _ids_full():
        pltpu.sync_copy(ids_hbm.at[pl.ds(base, _NPT + 16)], ids_v)

    @pl.when(sid == _NT - 1)
    def _ids_tail():
        real = _N - (_NT - 1) * _NPT
        pltpu.sync_copy(ids_hbm.at[pl.ds((_NT - 1) * _NPT, real)],
                        ids_v.at[pl.ds(0, real)])
        pltpu.sync_copy(tail_hbm, ids_v.at[pl.ds(real, _NTAIL)])

    zeros16 = jnp.zeros((16,), jnp.float32)

    def _zero(i, carry):
        acc_s[pl.ds(i * 16, 16)] = zeros16
        acc_c[pl.ds(i * 16, 16)] = zeros16
        return carry

    lax.fori_loop(0, _GP // 16, _zero, 0)

    lane = lax.iota(jnp.int32, 16)
    pos = lax.convert_element_type(lane, jnp.float32) + 1.0
    is15 = lane == 15

    def _step(j, carry):
        off = j * 16
        v = s_v[pl.ds(off, 16)]
        ids = ids_v[pl.ds(off, 16)]
        idn = ids_v[pl.ds(off + 1, 16)]
        c = jnp.cumsum(v)
        bnd = ids != idn
        m_add = bnd | is15
        m_sub = bnd & jnp.logical_not(is15)
        plsc.addupdate_scatter(acc_s, [ids], c, mask=m_add)
        plsc.addupdate_scatter(acc_s, [idn], -c, mask=m_sub)
        plsc.addupdate_scatter(acc_c, [ids], pos, mask=m_add)
        plsc.addupdate_scatter(acc_c, [idn], -pos, mask=m_sub)
        return carry

    lax.fori_loop(0, _NV, _step, 0)

    pubs = [pltpu.async_copy(acc_s, part_s_hbm.at[pl.ds(sid * _GP, _GP)], sem),
            pltpu.async_copy(acc_c, part_c_hbm.at[pl.ds(sid * _GP, _GP)], sem)]
    for p in pubs:
        p.wait()
    plsc.subcore_barrier()

    gb = sid * _BPT
    reads = [pltpu.async_copy(b_hbm, b_v, sem)]
    for t in range(_NT):
        reads.append(pltpu.async_copy(
            part_s_hbm.at[pl.ds(t * _GP + gb, _BPT)],
            buf_s.at[pl.ds(t * _BPT, _BPT)], sem))
        reads.append(pltpu.async_copy(
            part_c_hbm.at[pl.ds(t * _GP + gb, _BPT)],
            buf_c.at[pl.ds(t * _BPT, _BPT)], sem))
    for r in reads:
        r.wait()
    bvec = b_v[...]
    for k in range(_BPT // 16):
        ss = zeros16
        cc = zeros16
        for t in range(_NT):
            ss = ss + buf_s[pl.ds(t * _BPT + k * 16, 16)]
            cc = cc + buf_c[pl.ds(t * _BPT + k * 16, 16)]
        pred_v[pl.ds(k * 16, 16)] = ss / jnp.maximum(cc, 1.0) + bvec
    pltpu.sync_copy(pred_v, out_hbm.at[pl.ds(gb, _BPT)])


def _segment_mean_linear(s_pad, ids, tail, b16):
    mesh = plsc.VectorSubcoreMesh(
        core_axis_name="c", subcore_axis_name="s", num_cores=1)
    f = functools.partial(
        pl.kernel,
        mesh=mesh,
        compiler_params=pltpu.CompilerParams(needs_layout_passes=False),
        out_type=(
            jax.ShapeDtypeStruct((_G,), jnp.float32),
            jax.ShapeDtypeStruct((_NT * _GP,), jnp.float32),
            jax.ShapeDtypeStruct((_NT * _GP,), jnp.float32),
        ),
        scratch_types=[
            pltpu.VMEM((_NPT,), jnp.float32),
            pltpu.VMEM((_NPT + 16,), jnp.int32),
            pltpu.VMEM((_GP,), jnp.float32),
            pltpu.VMEM((_GP,), jnp.float32),
            pltpu.VMEM((_G,), jnp.float32),
            pltpu.VMEM((_G,), jnp.float32),
            pltpu.VMEM((_BPT,), jnp.float32),
            pltpu.VMEM((16,), jnp.float32),
            pltpu.SemaphoreType.DMA,
        ],
    )(_seg_body)
    return f(s_pad, ids, tail, b16)[0]


def kernel(x, batch, y, W, b):
    s_pad = _rowdot(x, W.reshape(1, _D)).reshape(_NPAD)
    ids = batch.astype(jnp.int32)
    tail = jnp.full((_NTAIL,), _G, jnp.int32)
    b16 = jnp.broadcast_to(b.reshape(1), (16,)).astype(jnp.float32)
    pred = _segment_mean_linear(s_pad, ids, tail, b16).reshape(_G, 1)
    return (pred, y)


# speedup vs baseline: 6.6729x; 1.1331x over previous
import functools

import jax
import jax.numpy as jnp
from jax import lax
from jax.experimental import pallas as pl
from jax.experimental.pallas import tpu as pltpu
from jax.experimental.pallas import tpu_sc as plsc

_N = 100000
_D = 128
_G = 1024


_TILE = 5120


def _rowdot_body(x_ref, w_ref, o_ref):
    o_ref[...] = jax.lax.dot_general(
        x_ref[...], w_ref[...], (((1,), (1,)), ((), ())),
        preferred_element_type=jnp.float32)[:, 0]


def _rowdot(x, w_row):
    return pl.pallas_call(
        _rowdot_body,
        grid=(_NPAD // _TILE,),
        in_specs=[
            pl.BlockSpec((_TILE, _D), lambda i: (i, 0)),
            pl.BlockSpec((1, _D), lambda i: (0, 0)),
        ],
        out_specs=pl.BlockSpec((_TILE,), lambda i: (i,)),
        out_shape=jax.ShapeDtypeStruct((_NPAD,), jnp.float32),
    )(x, w_row)


_NT = 16
_NPAD = 102400
_NPT = _NPAD // _NT
_NV = _NPT // 16
_GP = _G + 16
_BPT = _G // _NT


_NTAIL = _NPAD + 16 - _N


def _seg_body(s_hbm, ids_hbm, tail_hbm, b_hbm, out_hbm, part_s_hbm, part_c_hbm,
              s_v, ids_v, acc_s, acc_c, buf_s, buf_c, pred_v, b_v, sem):
    sid = lax.axis_index("s")
    base = sid * _NPT
    pltpu.sync_copy(s_hbm.at[pl.ds(base, _NPT)], s_v)

    @pl.when(sid < _NT - 1)
    def _ids_full():
        pltpu.sync_copy(ids_hbm.at[pl.ds(base, _NPT + 16)], ids_v)

    @pl.when(sid == _NT - 1)
    def _ids_tail():
        real = _N - (_NT - 1) * _NPT
        pltpu.sync_copy(ids_hbm.at[pl.ds((_NT - 1) * _NPT, real)],
                        ids_v.at[pl.ds(0, real)])
        pltpu.sync_copy(tail_hbm, ids_v.at[pl.ds(real, _NTAIL)])

    zeros16 = jnp.zeros((16,), jnp.float32)

    def _zero(i, carry):
        acc_s[pl.ds(i * 16, 16)] = zeros16
        acc_c[pl.ds(i * 16, 16)] = zeros16
        return carry

    lax.fori_loop(0, _GP // 16, _zero, 0)

    lane = lax.iota(jnp.int32, 16)
    pos = lax.convert_element_type(lane, jnp.float32) + 1.0
    is15 = lane == 15

    def _step(j, carry):
        off = j * 16
        v = s_v[pl.ds(off, 16)]
        ids = ids_v[pl.ds(off, 16)]
        idn = ids_v[pl.ds(off + 1, 16)]
        c = jnp.cumsum(v)
        bnd = ids != idn
        m_add = bnd | is15
        m_sub = bnd & jnp.logical_not(is15)
        plsc.addupdate_scatter(acc_s, [ids], c, mask=m_add)
        plsc.addupdate_scatter(acc_s, [idn], -c, mask=m_sub)
        plsc.addupdate_scatter(acc_c, [ids], pos, mask=m_add)
        plsc.addupdate_scatter(acc_c, [idn], -pos, mask=m_sub)
        return carry

    lax.fori_loop(0, _NV, _step, 0)

    pubs = [pltpu.async_copy(acc_s, part_s_hbm.at[pl.ds(sid * _GP, _GP)], sem),
            pltpu.async_copy(acc_c, part_c_hbm.at[pl.ds(sid * _GP, _GP)], sem)]
    for p in pubs:
        p.wait()
    plsc.subcore_barrier()

    gb = sid * _BPT
    reads = [pltpu.async_copy(b_hbm, b_v, sem)]
    for t in range(_NT):
        reads.append(pltpu.async_copy(
            part_s_hbm.at[pl.ds(t * _GP + gb, _BPT)],
            buf_s.at[pl.ds(t * _BPT, _BPT)], sem))
        reads.append(pltpu.async_copy(
            part_c_hbm.at[pl.ds(t * _GP + gb, _BPT)],
            buf_c.at[pl.ds(t * _BPT, _BPT)], sem))
    for r in reads:
        r.wait()
    bvec = b_v[...]
    for k in range(_BPT // 16):
        ss = zeros16
        cc = zeros16
        for t in range(_NT):
            ss = ss + buf_s[pl.ds(t * _BPT + k * 16, 16)]
            cc = cc + buf_c[pl.ds(t * _BPT + k * 16, 16)]
        pred_v[pl.ds(k * 16, 16)] = ss / jnp.maximum(cc, 1.0) + bvec
    pltpu.sync_copy(pred_v, out_hbm.at[pl.ds(gb, _BPT)])


def _segment_mean_linear(s_pad, ids, tail, b16):
    mesh = plsc.VectorSubcoreMesh(
        core_axis_name="c", subcore_axis_name="s", num_cores=1)
    f = functools.partial(
        pl.kernel,
        mesh=mesh,
        compiler_params=pltpu.CompilerParams(needs_layout_passes=False),
        out_type=(
            jax.ShapeDtypeStruct((_G,), jnp.float32),
            jax.ShapeDtypeStruct((_NT * _GP,), jnp.float32),
            jax.ShapeDtypeStruct((_NT * _GP,), jnp.float32),
        ),
        scratch_types=[
            pltpu.VMEM((_NPT,), jnp.float32),
            pltpu.VMEM((_NPT + 16,), jnp.int32),
            pltpu.VMEM((_GP,), jnp.float32),
            pltpu.VMEM((_GP,), jnp.float32),
            pltpu.VMEM((_G,), jnp.float32),
            pltpu.VMEM((_G,), jnp.float32),
            pltpu.VMEM((_BPT,), jnp.float32),
            pltpu.VMEM((16,), jnp.float32),
            pltpu.SemaphoreType.DMA,
        ],
    )(_seg_body)
    return f(s_pad, ids, tail, b16)[0]


def kernel(x, batch, y, W, b):
    s_pad = _rowdot(x, W.reshape(1, _D))
    ids = batch.astype(jnp.int32)
    tail = jnp.full((_NTAIL,), _G, jnp.int32)
    b16 = jnp.broadcast_to(b.reshape(1), (16,)).astype(jnp.float32)
    pred = _segment_mean_linear(s_pad, ids, tail, b16).reshape(_G, 1)
    return (pred, y)


# speedup vs baseline: 7.7360x; 1.1593x over previous
import functools

import jax
import jax.numpy as jnp
from jax import lax
from jax.experimental import pallas as pl
from jax.experimental.pallas import tpu as pltpu
from jax.experimental.pallas import tpu_sc as plsc

_N = 100000
_D = 128
_G = 1024

_CH = 128
_NFULL = _N // _CH
_TAILR = _N - _NFULL * _CH
_NW = 32
_ROUNDS = _NFULL // _NW
_EXTRA = _NFULL - _ROUNDS * _NW
_GP = _G + 16
_BPT = _G // 16


def _count_vectors(ids_v, acc_c, nvec, pos, is15):
    def _step(j, carry):
        off = j * 16
        ids = ids_v[pl.ds(off, 16)]
        idn = ids_v[pl.ds(off + 1, 16)]
        bnd = ids != idn
        m_add = bnd | is15
        m_sub = bnd & jnp.logical_not(is15)
        plsc.addupdate_scatter(acc_c, [ids], pos, mask=m_add)
        plsc.addupdate_scatter(acc_c, [idn], -pos, mask=m_sub)
        return carry
    lax.fori_loop(0, nvec, _step, 0)


def _seg_body(x_hbm, ids_hbm, z_hbm, sums_hbm, part_c_hbm,
              xb0, xb1, id0, id1, idt, idc0, idc1, acc_c, acc_sh,
              sem0, sem1):
    cid = lax.axis_index("c")
    sid = lax.axis_index("s")
    w = sid * 2 + cid

    pltpu.sync_copy(z_hbm.at[pl.ds(sid * _BPT, _BPT), :],
                    acc_sh.at[pl.ds(sid * _BPT, _BPT), :])
    z16 = jnp.zeros((16,), jnp.float32)

    def _zero(i, carry):
        acc_c[pl.ds(i * 16, 16)] = z16
        return carry
    lax.fori_loop(0, _GP // 16, _zero, 0)

    lane = lax.iota(jnp.int32, 16)
    pos = lax.convert_element_type(lane, jnp.float32) + 1.0
    is15 = lane == 15
    plsc.subcore_barrier()

    bufs = ((xb0, id0, idc0, sem0), (xb1, id1, idc1, sem1))

    def _fetch(chunk, slot):
        xb, idv, idc, sem = bufs[slot]
        base = chunk * _CH
        return (pltpu.async_copy(x_hbm.at[pl.ds(base, _CH), :], xb, sem),
                pltpu.async_copy(ids_hbm.at[pl.ds(base, _CH)], idv, sem),
                pltpu.async_copy(ids_hbm.at[pl.ds(base, _CH + 16)], idc, sem))

    def _consume(slot):
        xb, idv, idc, _ = bufs[slot]
        pltpu.sync_copy(xb, acc_sh.at[idv], add=True)
        _count_vectors(idc, acc_c, _CH // 16, pos, is15)

    cps = _fetch(w, 0)
    for r in range(_ROUNDS):
        nxt = None
        if r + 1 < _ROUNDS:
            nxt = _fetch((r + 1) * _NW + w, (r + 1) % 2)
        for c in cps:
            c.wait()
        _consume(r % 2)
        cps = nxt

    @pl.when(w < _EXTRA)
    def _extra():
        chunk = _ROUNDS * _NW + w
        base = chunk * _CH
        e = (pltpu.async_copy(x_hbm.at[pl.ds(base, _CH), :], xb0, sem0),
             pltpu.async_copy(ids_hbm.at[pl.ds(base, _CH)], id0, sem0),
             pltpu.async_copy(ids_hbm.at[pl.ds(base, _CH + 16)], idc0, sem0))
        for c in e:
            c.wait()
        pltpu.sync_copy(xb0, acc_sh.at[id0], add=True)
        _count_vectors(idc0, acc_c, _CH // 16, pos, is15)

    @pl.when(w == _EXTRA)
    def _tail():
        base = _NFULL * _CH
        e = (pltpu.async_copy(x_hbm.at[pl.ds(base, _TAILR), :],
                              xb1.at[pl.ds(0, _TAILR), :], sem1),
             pltpu.async_copy(ids_hbm.at[pl.ds(base, _TAILR)], idt, sem1),
             pltpu.async_copy(ids_hbm.at[pl.ds(base, _TAILR + 16)],
                              idc1.at[pl.ds(0, _TAILR + 16)], sem1))
        for c in e:
            c.wait()
        pltpu.sync_copy(xb1.at[pl.ds(0, _TAILR), :], acc_sh.at[idt], add=True)
        _count_vectors(idc1, acc_c, _TAILR // 16, pos, is15)

    pltpu.sync_copy(acc_c, part_c_hbm.at[pl.ds(w * _GP, _GP)])
    plsc.subcore_barrier()

    pltpu.sync_copy(acc_sh.at[pl.ds(sid * _BPT, _BPT), :],
                    sums_hbm.at[pl.ds(cid * _G + sid * _BPT, _BPT), :])


def _sc_segment_sum(x, ids, zeros_rows):
    mesh = plsc.VectorSubcoreMesh(core_axis_name="c", subcore_axis_name="s")
    f = functools.partial(
        pl.kernel,
        mesh=mesh,
        compiler_params=pltpu.CompilerParams(needs_layout_passes=False),
        out_type=(
            jax.ShapeDtypeStruct((2 * _G, _D), jnp.float32),
            jax.ShapeDtypeStruct((_NW * _GP,), jnp.float32),
        ),
        scratch_types=[
            pltpu.VMEM((_CH, _D), jnp.float32),
            pltpu.VMEM((_CH, _D), jnp.float32),
            pltpu.VMEM((_CH,), jnp.int32),
            pltpu.VMEM((_CH,), jnp.int32),
            pltpu.VMEM((_TAILR,), jnp.int32),
            pltpu.VMEM((_CH + 16,), jnp.int32),
            pltpu.VMEM((_CH + 16,), jnp.int32),
            pltpu.VMEM((_GP,), jnp.float32),
            pltpu.VMEM_SHARED((_G, _D), jnp.float32),
            pltpu.SemaphoreType.DMA,
            pltpu.SemaphoreType.DMA,
        ],
    )(_seg_body)
    return f(x, ids, zeros_rows)


def _fin_body(sums_ref, cnt_ref, w_ref, b_ref, o_ref):
    s = sums_ref[pl.ds(0, _G), :] + sums_ref[pl.ds(_G, _G), :]
    row = jax.lax.dot_general(
        w_ref[...], s, (((1,), (1,)), ((), ())),
        preferred_element_type=jnp.float32)
    cnt = jnp.zeros((_G,), jnp.float32)
    for t in range(_NW):
        cnt = cnt + cnt_ref[pl.ds(t * _GP, _G)]
    pred = row[0] / jnp.maximum(cnt, 1.0) + b_ref[0, 0]
    o_ref[...] = pred.reshape(_G, 1)


def _finalize(sums2, part_c, w_row, b):
    return pl.pallas_call(
        _fin_body,
        out_shape=jax.ShapeDtypeStruct((_G, 1), jnp.float32),
    )(sums2, part_c, w_row, b.reshape(1, 1))


def kernel(x, batch, y, W, b):
    ids = jnp.concatenate(
        [batch.astype(jnp.int32), jnp.full((16,), _G, jnp.int32)])
    zeros_rows = jnp.zeros((_G, _D), jnp.float32)
    sums2, part_c = _sc_segment_sum(x, ids, zeros_rows)
    pred = _finalize(sums2, part_c, W.reshape(1, _D), b)
    return (pred, y)
